# baseline (speedup 1.0000x reference)
import jax
import jax.numpy as jnp
from jax.experimental import pallas as pl
from jax.experimental.pallas import tpu as pltpu

_B, _F, _P, _N = 64, 16, 576, 24
_BB = 8


def _in_prior(r, c, valid):
    return valid & (r >= 4) & (c >= 2) & (c <= _N - 3)


def _body(score_ref, out_ref):
    s = score_ref[...].reshape(_BB * _F, _P)
    iota2 = jax.lax.broadcasted_iota(jnp.int32, (_BB * _F, _P), 1)
    m = jnp.max(s, axis=1, keepdims=True)
    qbf = jnp.min(jnp.where(s == m, iota2, _P), axis=1).reshape(_BB, _F)

    qfb = qbf.T
    r = qfb // _N
    c = qfb % _N
    pn = (_in_prior(r - 1, c, r >= 1) | _in_prior(r + 1, c, r <= _N - 2)
          | _in_prior(r, c - 1, c >= 1) | _in_prior(r, c + 1, c <= _N - 2))
    dr = jnp.abs(r[:, None, :] - r[None, :, :])
    dc = jnp.abs(c[:, None, :] - c[None, :, :])
    adjm = (dr + dc) == 1
    fio = jax.lax.broadcasted_iota(jnp.int32, (_F, _BB), 0)
    add = pn
    for i in range(1, _F):
        contrib = jnp.any(add & (fio < i) & adjm[i], axis=0, keepdims=True)
        add = add | ((fio == i) & contrib)
    addbf = add.T

    patch_iota = jax.lax.broadcasted_iota(jnp.int32, (_BB, _P), 1)
    col = patch_iota % _N
    b = jnp.where((patch_iota >= 4 * _N) & (col >= 2) & (col < _N - 2),
                  1.0, 0.0).astype(jnp.float32)
    out_ref[:, 0:1] = jnp.ones((_BB, 1), jnp.float32)
    for i in range(_F):
        qi = qbf[:, i].reshape(_BB, 1)
        addi = addbf[:, i].reshape(_BB, 1)
        newbit = jnp.where((patch_iota == qi) & addi, 1.0, 0.0)
        b = jnp.maximum(b, newbit)
        out_ref[:, 1 + _P * i : 1 + _P * (i + 1)] = b


@jax.jit
def kernel(score):
    return pl.pallas_call(
        _body,
        grid=(_B // _BB,),
        in_specs=[pl.BlockSpec((_BB, _F, _P), lambda i: (i, 0, 0))],
        out_specs=pl.BlockSpec((_BB, 1 + _F * _P), lambda i: (i, 0)),
        out_shape=jax.ShapeDtypeStruct((_B, 1 + _F * _P), jnp.float32),
        compiler_params=pltpu.CompilerParams(
            dimension_semantics=("parallel",)),
    )(score)


# speedup vs baseline: 2.0269x; 2.0269x over previous
import jax
import jax.numpy as jnp
from jax.experimental import pallas as pl
from jax.experimental.pallas import tpu as pltpu

_B, _F, _P, _N = 64, 16, 576, 24
_BB = 32


def _in_prior(r, c, valid):
    return valid & (r >= 4) & (c >= 2) & (c <= _N - 3)


def _body(score_ref, out_ref):
    s = score_ref[...].reshape(_BB * _F, _P)
    iota2 = jax.lax.broadcasted_iota(jnp.int32, (_BB * _F, _P), 1)
    m = jnp.max(s, axis=1, keepdims=True)
    qbf = jnp.min(jnp.where(s == m, iota2, _P), axis=1).reshape(_BB, _F)

    qfb = qbf.T
    r = qfb // _N
    c = qfb % _N
    pn = (_in_prior(r - 1, c, r >= 1) | _in_prior(r + 1, c, r <= _N - 2)
          | _in_prior(r, c - 1, c >= 1) | _in_prior(r, c + 1, c <= _N - 2))
    dr = jnp.abs(r[:, None, :] - r[None, :, :])
    dc = jnp.abs(c[:, None, :] - c[None, :, :])
    adjm = (dr + dc) == 1
    fio = jax.lax.broadcasted_iota(jnp.int32, (_F, _BB), 0)
    add = pn
    for i in range(1, _F):
        contrib = jnp.any(add & (fio < i) & adjm[i], axis=0, keepdims=True)
        add = add | ((fio == i) & contrib)
    qeff = jnp.where(add, qfb, -1).T

    patch_iota = jax.lax.broadcasted_iota(jnp.int32, (_BB, _P), 1)
    col = patch_iota % _N
    b = jnp.where((patch_iota >= 4 * _N) & (col >= 2) & (col < _N - 2),
                  1.0, 0.0).astype(jnp.float32)
    out_ref[:, 0:1] = jnp.ones((_BB, 1), jnp.float32)
    for i in range(_F):
        qi = qeff[:, i].reshape(_BB, 1)
        b = jnp.where(patch_iota == qi, 1.0, b)
        out_ref[:, 1 + _P * i : 1 + _P * (i + 1)] = b


@jax.jit
def kernel(score):
    return pl.pallas_call(
        _body,
        grid=(_B // _BB,),
        in_specs=[pl.BlockSpec((_BB, _F, _P), lambda i: (i, 0, 0))],
        out_specs=pl.BlockSpec((_BB, 1 + _F * _P), lambda i: (i, 0)),
        out_shape=jax.ShapeDtypeStruct((_B, 1 + _F * _P), jnp.float32),
        compiler_params=pltpu.CompilerParams(
            dimension_semantics=("parallel",)),
    )(score)
